# baseline (speedup 1.0000x reference)
import functools

import jax
import jax.numpy as jnp
from jax import lax
from jax.experimental import pallas as pl
from jax.experimental.pallas import tpu as pltpu
from jax.experimental.pallas import tpu_sc as plsc

NC = 2
NS = 16
BATCH = 128

_REL_LIST = [
    ("atom", "atom"), ("atom", "bond"), ("atom", "motif"),
    ("bond", "bond"), ("bond", "motif"), ("motif", "motif"),
    ("atom", "cell"), ("bond", "cell"), ("motif", "cell"),
]
_DST_RELS = {
    "atom": (0,), "bond": (1, 3), "motif": (2, 4, 5), "cell": (6, 7, 8),
}


def _ndp(n):
    return (n // 128 + 1) * 128


GB = 384
KI = 6


@functools.lru_cache(maxsize=None)
def _segsum_kernel(nsuper, ndp, two_ns):
    rstripe = ndp // NS
    mesh = plsc.VectorSubcoreMesh(
        core_axis_name="c", subcore_axis_name="s",
        num_cores=NC, num_subcores=NS)

    @functools.partial(
        pl.kernel,
        out_type=jax.ShapeDtypeStruct((NC, ndp, 32), jnp.float32),
        mesh=mesh,
        scratch_types=[
            pltpu.VMEM((KI, GB), jnp.int32),
            pltpu.VMEM((KI, GB), jnp.int32),
            pltpu.VMEM((2, GB, 32), jnp.float32),
            pltpu.VMEM_SHARED((ndp, 32), jnp.float32),
            pltpu.SemaphoreType.DMA,
            pltpu.SemaphoreType.DMA,
            pltpu.SemaphoreType.DMA,
        ],
        compiler_params=pltpu.CompilerParams(use_tc_tiling_on_sc=False),
    )
    def seg(xflat, srcp, dst, zeros, drain, out,
            src_v, dst_v, rows_v, acc_sh, sem_s, g0, g1):
        c = lax.axis_index("c")
        s = lax.axis_index("s")
        gsems = (g0, g1)
        off = pl.multiple_of(s * rstripe, 8)
        pltpu.sync_copy(zeros, acc_sh.at[pl.ds(off, rstripe)])
        plsc.subcore_barrier()

        def outer(g, carry):
            pltpu.sync_copy(srcp.at[c].at[s].at[g], src_v)
            pltpu.sync_copy(dst.at[s].at[g], dst_v)
            gd = {}
            for j in range(min(2, KI)):
                gd[j] = pltpu.async_copy(xflat.at[src_v.at[j]],
                                         rows_v.at[j % 2], gsems[j % 2])
            for j in range(KI):
                gd[j].wait()
                pltpu.async_copy(rows_v.at[j % 2], acc_sh.at[dst_v.at[j]],
                                 sem_s, add=True)
                nj = j + 2
                if nj < KI:
                    pltpu.make_async_copy(drain.at[0], rows_v.at[0],
                                          sem_s).wait()
                    gd[nj] = pltpu.async_copy(xflat.at[src_v.at[nj]],
                                              rows_v.at[nj % 2],
                                              gsems[nj % 2])
            pltpu.make_async_copy(drain, rows_v, sem_s).wait()
            return carry

        lax.fori_loop(0, nsuper, outer, 0)
        plsc.subcore_barrier()
        pltpu.sync_copy(acc_sh.at[pl.ds(off, rstripe)],
                        out.at[c].at[pl.ds(off, rstripe)])

    return seg


def _prep_edges(src, dst, n_src, ndp):
    e = src.shape[0]
    nsuper = -(-e // (NS * KI * GB))
    ep = NS * KI * GB * nsuper
    if ep > e:
        src = jnp.concatenate([src, jnp.zeros((ep - e,), jnp.int32)])
        dst = jnp.concatenate([dst, jnp.full((ep - e,), ndp - 1, jnp.int32)])
    src_rs = src.reshape(NS, nsuper, KI, GB)
    srcp = jnp.stack([src_rs, src_rs + n_src])
    return srcp, dst.reshape(NS, nsuper, KI, GB), nsuper


@functools.lru_cache(maxsize=None)
def _dense_kernel(n, ndp, br, nr, final):
    grid = (n // br,)
    f32 = jnp.float32

    def body(*refs):
        xh = refs[0]
        srefs = refs[1:1 + nr]
        crefs = refs[1 + nr:1 + 2 * nr]
        wl, wr, b = refs[1 + 2 * nr:4 + 2 * nr]
        out = refs[-1]
        acc = (jnp.dot(xh[0], wr[:32, :], preferred_element_type=f32)
               + jnp.dot(xh[1], wr[32:, :], preferred_element_type=f32)
               + b[...])
        for r in range(nr):
            rc = 1.0 / jnp.maximum(crefs[r][...], 1.0)
            acc += jnp.dot(srefs[r][0] * rc, wl[r, :32, :],
                           preferred_element_type=f32)
            acc += jnp.dot(srefs[r][1] * rc, wl[r, 32:, :],
                           preferred_element_type=f32)
        y = jnp.maximum(acc, 0.0)
        if final:
            pw, pb, ow, ob = refs[4 + 2 * nr:8 + 2 * nr]
            h = jnp.dot(y, pw[...], preferred_element_type=f32) + pb[...]
            h = jax.nn.softplus(h)
            out[...] = jnp.dot(h, ow[...], preferred_element_type=f32) + ob[...]
        else:
            out[0] = y[:, :32]
            out[1] = y[:, 32:]

    full = lambda shape: pl.BlockSpec(shape, lambda i: (0,) * len(shape))
    in_specs = [pl.BlockSpec((NC, br, 32), lambda i: (0, i, 0))]
    in_specs += [pl.BlockSpec((NC, br, 32), lambda i: (0, i, 0))] * nr
    in_specs += [pl.BlockSpec((br, 32), lambda i: (i, 0))] * nr
    in_specs += [full((nr, 64, 64)), full((64, 64)), full((1, 64))]
    if final:
        in_specs += [full((64, 64)), full((1, 64)), full((64, 1)),
                     full((1, 1))]
        out_spec = pl.BlockSpec((br, 1), lambda i: (i, 0))
        out_shape = jax.ShapeDtypeStruct((n, 1), f32)
    else:
        out_spec = pl.BlockSpec((NC, br, 32), lambda i: (0, i, 0))
        out_shape = jax.ShapeDtypeStruct((NC, n, 32), f32)

    return pl.pallas_call(body, grid=grid, in_specs=in_specs,
                          out_specs=out_spec, out_shape=out_shape)


def kernel(x_atom, x_bond, x_motif, x_cell, e_atom_bonds_atom, e_atom_in_bond,
           e_atom_in_motif, e_bond_touches_bond, e_bond_in_motif,
           e_motif_touches_motif, e_atom_in_cell, e_bond_in_cell,
           e_motif_in_cell, W_l, b_l, W_r, proj_W, proj_b, out_W, out_b):
    xs = {"atom": x_atom, "bond": x_bond, "motif": x_motif, "cell": x_cell}
    edges = [e_atom_bonds_atom, e_atom_in_bond, e_atom_in_motif,
             e_bond_touches_bond, e_bond_in_motif, e_motif_touches_motif,
             e_atom_in_cell, e_bond_in_cell, e_motif_in_cell]
    nn = {t: x.shape[0] for t, x in xs.items()}
    ndp = {t: _ndp(n) for t, n in nn.items()}
    zeros = {t: jnp.zeros((ndp[t] // NS, 32), jnp.float32) for t in xs}
    drain = jnp.zeros((2, GB, 32), jnp.float32)

    xh = {t: jnp.stack([x[:, :32], x[:, 32:]]) for t, x in xs.items()}

    prep, counts = [], []
    ones_tab = jnp.ones((2, 32), jnp.float32)
    for i, (s, d) in enumerate(_REL_LIST):
        srcp, dstp, k = _prep_edges(edges[i][0], edges[i][1], nn[s], ndp[d])
        prep.append((srcp, dstp, k))
        csrc, _, _ = _prep_edges(jnp.zeros_like(edges[i][0]), edges[i][1],
                                 1, ndp[d])
        cnt = _segsum_kernel(k, ndp[d], 2)(ones_tab, csrc, dstp, zeros[d],
                                           drain)
        counts.append(cnt[0])

    br = {"atom": 1000, "bond": 1000, "motif": 1000, "cell": 1000}

    for layer in range(3):
        live = ("cell",) if layer == 2 else ("atom", "bond", "motif", "cell")
        sums = {}
        for i, (s, d) in enumerate(_REL_LIST):
            if d not in live:
                continue
            srcp, dstp, k = prep[i]
            xflat = xh[s].reshape(2 * nn[s], 32)
            sums[i] = _segsum_kernel(k, ndp[d], 2 * nn[s])(
                xflat, srcp, dstp, zeros[d], drain)
        new_xh = {}
        for d in live:
            rels = _DST_RELS[d]
            nr = len(rels)
            wl = jnp.stack([W_l[layer, i] for i in rels])
            wr = sum(W_r[layer, i] for i in rels)
            b = sum(b_l[layer, i] for i in rels).reshape(1, 64)
            final = layer == 2
            args = ([xh[d]] + [sums[i] for i in rels]
                    + [counts[i] for i in rels] + [wl, wr, b])
            if final:
                args += [proj_W, proj_b.reshape(1, 64), out_W,
                         out_b.reshape(1, 1)]
            res = _dense_kernel(nn[d], ndp[d], br[d], nr, final)(*args)
            if final:
                return res
            new_xh[d] = res
        xh = new_xh


# speedup vs baseline: 6.4603x; 6.4603x over previous
import functools

import jax
import jax.numpy as jnp
from jax import lax
from jax.experimental import pallas as pl
from jax.experimental.pallas import tpu as pltpu
from jax.experimental.pallas import tpu_sc as plsc

NC = 2
NS = 16
BATCH = 128

_REL_LIST = [
    ("atom", "atom"), ("atom", "bond"), ("atom", "motif"),
    ("bond", "bond"), ("bond", "motif"), ("motif", "motif"),
    ("atom", "cell"), ("bond", "cell"), ("motif", "cell"),
]
_DST_RELS = {
    "atom": (0,), "bond": (1, 3), "motif": (2, 4, 5), "cell": (6, 7, 8),
}


def _ndp(n):
    return (n // 128 + 1) * 128


KC = 12
RING = 6
GBC = 512
KI2 = 8


@functools.lru_cache(maxsize=None)
def _segsum_kernel(nsuper, ndp, two_ns):
    rstripe = ndp // NS
    mesh = plsc.VectorSubcoreMesh(
        core_axis_name="c", subcore_axis_name="s",
        num_cores=NC, num_subcores=NS)

    @functools.partial(
        pl.kernel,
        out_type=jax.ShapeDtypeStruct((NC, ndp, 32), jnp.float32),
        mesh=mesh,
        scratch_types=[
            pltpu.VMEM((2, KC, BATCH), jnp.int32),
            pltpu.VMEM((RING, BATCH, 32), jnp.float32),
            pltpu.VMEM_SHARED((ndp, 32), jnp.float32),
            pltpu.SemaphoreType.DMA,
        ] + [pltpu.SemaphoreType.DMA] * RING,
        compiler_params=pltpu.CompilerParams(use_tc_tiling_on_sc=False),
    )
    def seg(xflat, sd, zeros, drain, out, sd_v, rows_v, acc_sh, sem_s, *gsems):
        c = lax.axis_index("c")
        s = lax.axis_index("s")
        off = pl.multiple_of(s * rstripe, 8)
        pltpu.sync_copy(zeros, acc_sh.at[pl.ds(off, rstripe)])
        plsc.subcore_barrier()
        src_v = sd_v.at[0]
        dst_v = sd_v.at[1]

        def outer(g, carry):
            pltpu.sync_copy(sd.at[c].at[s].at[g], sd_v)
            gd = {}
            for j in range(min(RING, KC)):
                gd[j] = pltpu.async_copy(xflat.at[src_v.at[j]],
                                         rows_v.at[j % RING], gsems[j % RING])
            for j in range(KC):
                gd[j].wait()
                pltpu.async_copy(rows_v.at[j % RING], acc_sh.at[dst_v.at[j]],
                                 sem_s, add=True)
                nj = j + RING
                if nj < KC:
                    pltpu.make_async_copy(drain.at[0], rows_v.at[0],
                                          sem_s).wait()
                    gd[nj] = pltpu.async_copy(xflat.at[src_v.at[nj]],
                                              rows_v.at[nj % RING],
                                              gsems[nj % RING])
            pltpu.make_async_copy(drain, rows_v, sem_s).wait()
            return carry

        lax.fori_loop(0, nsuper, outer, 0)
        plsc.subcore_barrier()
        pltpu.sync_copy(acc_sh.at[pl.ds(off, rstripe)],
                        out.at[c].at[pl.ds(off, rstripe)])

    return seg


@functools.lru_cache(maxsize=None)
def _counts_kernel(nsuper, ndp):
    rstripe = ndp // NS
    mesh = plsc.VectorSubcoreMesh(
        core_axis_name="c", subcore_axis_name="s",
        num_cores=NC, num_subcores=NS)

    @functools.partial(
        pl.kernel,
        out_type=jax.ShapeDtypeStruct((NC, ndp, 16), jnp.float32),
        mesh=mesh,
        scratch_types=[
            pltpu.VMEM((KI2, GBC), jnp.int32),
            pltpu.VMEM((GBC, 16), jnp.float32),
            pltpu.VMEM_SHARED((ndp, 16), jnp.float32),
            pltpu.SemaphoreType.DMA,
        ],
        compiler_params=pltpu.CompilerParams(use_tc_tiling_on_sc=False),
    )
    def cnt(dst, ones, zeros16, out, dst_v, ones_v, acc_sh, sem_s):
        c = lax.axis_index("c")
        s = lax.axis_index("s")
        off = pl.multiple_of(s * rstripe, 8)
        pltpu.sync_copy(zeros16, acc_sh.at[pl.ds(off, rstripe)])
        pltpu.sync_copy(ones, ones_v)
        plsc.subcore_barrier()

        def outer(g, carry):
            pltpu.sync_copy(dst.at[c].at[s].at[g], dst_v)
            for j in range(KI2):
                pltpu.async_copy(ones_v, acc_sh.at[dst_v.at[j]],
                                 sem_s, add=True)
            for _ in range(KI2):
                pltpu.make_async_copy(ones, ones_v, sem_s).wait()
            return carry

        lax.fori_loop(0, nsuper, outer, 0)
        plsc.subcore_barrier()
        pltpu.sync_copy(acc_sh.at[pl.ds(off, rstripe)],
                        out.at[c].at[pl.ds(off, rstripe)])

    return cnt


def _prep_edges(src, dst, n_src, ndp):
    e = src.shape[0]
    nsuper = -(-e // (NS * KC * BATCH))
    ep = NS * KC * BATCH * nsuper
    if ep > e:
        src = jnp.concatenate([src, jnp.zeros((ep - e,), jnp.int32)])
        dst = jnp.concatenate([dst, jnp.full((ep - e,), ndp - 1, jnp.int32)])
    src_rs = src.reshape(NS, nsuper, KC, BATCH)
    dst_rs = dst.reshape(NS, nsuper, KC, BATCH)
    sd = jnp.stack([jnp.stack([src_rs + c * n_src, dst_rs], axis=2)
                    for c in range(NC)])
    return sd, nsuper


def _prep_counts(dst, ndp):
    e = dst.shape[0]
    nsuper = -(-e // (NC * NS * KI2 * GBC))
    ep = NC * NS * KI2 * GBC * nsuper
    if ep > e:
        dst = jnp.concatenate([dst, jnp.full((ep - e,), ndp - 1, jnp.int32)])
    return dst.reshape(NC, NS, nsuper, KI2, GBC), nsuper


@functools.lru_cache(maxsize=None)
def _dense_kernel(n, ndp, br, nr, final):
    grid = (n // br,)
    f32 = jnp.float32

    def body(*refs):
        xh = refs[0]
        srefs = refs[1:1 + nr]
        crefs = refs[1 + nr:1 + 2 * nr]
        wl, wr, b = refs[1 + 2 * nr:4 + 2 * nr]
        out = refs[-1]
        acc = (jnp.dot(xh[0], wr[:32, :], preferred_element_type=f32)
               + jnp.dot(xh[1], wr[32:, :], preferred_element_type=f32)
               + b[...])
        for r in range(nr):
            cr = crefs[r][0, :, :1] + crefs[r][1, :, :1]
            rc = 1.0 / jnp.maximum(cr, 1.0)
            acc += jnp.dot(srefs[r][0] * rc, wl[r, :32, :],
                           preferred_element_type=f32)
            acc += jnp.dot(srefs[r][1] * rc, wl[r, 32:, :],
                           preferred_element_type=f32)
        y = jnp.maximum(acc, 0.0)
        if final:
            pw, pb, ow, ob = refs[4 + 2 * nr:8 + 2 * nr]
            h = jnp.dot(y, pw[...], preferred_element_type=f32) + pb[...]
            h = jax.nn.softplus(h)
            out[...] = jnp.dot(h, ow[...], preferred_element_type=f32) + ob[...]
        else:
            out[0] = y[:, :32]
            out[1] = y[:, 32:]

    full = lambda shape: pl.BlockSpec(shape, lambda i: (0,) * len(shape))
    in_specs = [pl.BlockSpec((NC, br, 32), lambda i: (0, i, 0))]
    in_specs += [pl.BlockSpec((NC, br, 32), lambda i: (0, i, 0))] * nr
    in_specs += [pl.BlockSpec((NC, br, 16), lambda i: (0, i, 0))] * nr
    in_specs += [full((nr, 64, 64)), full((64, 64)), full((1, 64))]
    if final:
        in_specs += [full((64, 64)), full((1, 64)), full((64, 1)),
                     full((1, 1))]
        out_spec = pl.BlockSpec((br, 1), lambda i: (i, 0))
        out_shape = jax.ShapeDtypeStruct((n, 1), f32)
    else:
        out_spec = pl.BlockSpec((NC, br, 32), lambda i: (0, i, 0))
        out_shape = jax.ShapeDtypeStruct((NC, n, 32), f32)

    return pl.pallas_call(body, grid=grid, in_specs=in_specs,
                          out_specs=out_spec, out_shape=out_shape)


def kernel(x_atom, x_bond, x_motif, x_cell, e_atom_bonds_atom, e_atom_in_bond,
           e_atom_in_motif, e_bond_touches_bond, e_bond_in_motif,
           e_motif_touches_motif, e_atom_in_cell, e_bond_in_cell,
           e_motif_in_cell, W_l, b_l, W_r, proj_W, proj_b, out_W, out_b):
    xs = {"atom": x_atom, "bond": x_bond, "motif": x_motif, "cell": x_cell}
    edges = [e_atom_bonds_atom, e_atom_in_bond, e_atom_in_motif,
             e_bond_touches_bond, e_bond_in_motif, e_motif_touches_motif,
             e_atom_in_cell, e_bond_in_cell, e_motif_in_cell]
    nn = {t: x.shape[0] for t, x in xs.items()}
    ndp = {t: _ndp(n) for t, n in nn.items()}
    zeros = {t: jnp.zeros((ndp[t] // NS, 32), jnp.float32) for t in xs}
    zeros16 = {t: jnp.zeros((ndp[t] // NS, 16), jnp.float32) for t in xs}
    drain = jnp.zeros((RING, BATCH, 32), jnp.float32)
    ones16 = jnp.ones((GBC, 16), jnp.float32)

    xh = {t: jnp.stack([x[:, :32], x[:, 32:]]) for t, x in xs.items()}

    prep, counts = [], []
    for i, (s, d) in enumerate(_REL_LIST):
        sd, nsuper = _prep_edges(edges[i][0], edges[i][1], nn[s], ndp[d])
        prep.append((sd, nsuper))
        cdst, cns = _prep_counts(edges[i][1], ndp[d])
        counts.append(_counts_kernel(cns, ndp[d])(cdst, ones16, zeros16[d]))

    br = {"atom": 1000, "bond": 1000, "motif": 1000, "cell": 1000}

    for layer in range(3):
        live = ("cell",) if layer == 2 else ("atom", "bond", "motif", "cell")
        sums = {}
        for i, (s, d) in enumerate(_REL_LIST):
            if d not in live:
                continue
            sd, nsuper = prep[i]
            xflat = xh[s].reshape(2 * nn[s], 32)
            sums[i] = _segsum_kernel(nsuper, ndp[d], 2 * nn[s])(
                xflat, sd, zeros[d], drain)
        new_xh = {}
        for d in live:
            rels = _DST_RELS[d]
            nr = len(rels)
            wl = jnp.stack([W_l[layer, i] for i in rels])
            wr = sum(W_r[layer, i] for i in rels)
            b = sum(b_l[layer, i] for i in rels).reshape(1, 64)
            final = layer == 2
            args = ([xh[d]] + [sums[i] for i in rels]
                    + [counts[i] for i in rels] + [wl, wr, b])
            if final:
                args += [proj_W, proj_b.reshape(1, 64), out_W,
                         out_b.reshape(1, 1)]
            res = _dense_kernel(nn[d], ndp[d], br[d], nr, final)(*args)
            if final:
                return res
            new_xh[d] = res
        xh = new_xh
